# baseline (speedup 1.0000x reference)
import functools

import jax
import jax.numpy as jnp
from jax import lax
from jax.experimental import pallas as pl
from jax.experimental.pallas import tpu as pltpu
from jax.experimental.pallas import tpu_sc as plsc

N = 10000
NPAD = 10240
D = 256
DH = 128
E = 160000
EPAD = 163840
NC = 2
NS = 16

_A_ROWS = EPAD // (NC * NS) // 128

_sc_mesh = plsc.VectorSubcoreMesh(core_axis_name="c", subcore_axis_name="s")


@functools.partial(
    pl.kernel,
    out_type=jax.ShapeDtypeStruct((NC, NPAD), jnp.float32),
    mesh=_sc_mesh,
    scratch_types=[
        pltpu.MemorySpace.VMEM((_A_ROWS, 128), jnp.int32),
        pltpu.MemorySpace.VMEM((128,), jnp.float32),
        pltpu.MemorySpace.VMEM((NPAD // NS,), jnp.float32),
        pltpu.MemorySpace.VMEM_SHARED((NPAD,), jnp.float32),
        pltpu.SemaphoreType.DMA,
    ],
)
def _deg_kernel(dst2d, out, idx_v, ones_v, zero_v, hist_sh, sem):
    c = lax.axis_index("c")
    s = lax.axis_index("s")
    zvec = jnp.zeros((16,), jnp.float32)
    for i in range(NPAD // NS // 16):
        zero_v[pl.ds(i * 16, 16)] = zvec
    ovec = jnp.ones((16,), jnp.float32)
    for i in range(8):
        ones_v[pl.ds(i * 16, 16)] = ovec
    pltpu.sync_copy(zero_v, hist_sh.at[pl.ds(s * (NPAD // NS), NPAD // NS)])
    plsc.subcore_barrier()
    row0 = (c * NS + s) * _A_ROWS
    pltpu.sync_copy(dst2d.at[pl.ds(row0, _A_ROWS)], idx_v)
    descs = []
    for j in range(_A_ROWS):
        descs.append(
            pltpu.async_copy(ones_v, hist_sh.at[idx_v.at[j]], sem, add=True)
        )
    for d in descs:
        d.wait()
    plsc.subcore_barrier()
    pltpu.sync_copy(
        hist_sh.at[pl.ds(s * (NPAD // NS), NPAD // NS)],
        out.at[c, pl.ds(s * (NPAD // NS), NPAD // NS)],
    )


_B_BLK = 1000


def _matmul_body(x_ref, w_ref, hist_ref, g_ref):
    h = jnp.dot(x_ref[...], w_ref[...], preferred_element_type=jnp.float32)
    deg = 1.0 + hist_ref[:, 0] + hist_ref[:, 1]
    dinv = lax.rsqrt(deg)[:, None]
    g_ref[0] = h[:, :DH] * dinv
    g_ref[1] = h[:, DH:] * dinv


def _matmul_scale(x, W, hist_t):
    return pl.pallas_call(
        _matmul_body,
        grid=(N // _B_BLK,),
        in_specs=[
            pl.BlockSpec((_B_BLK, D), lambda i: (i, 0)),
            pl.BlockSpec((D, D), lambda i: (0, 0)),
            pl.BlockSpec((_B_BLK, NC), lambda i: (i, 0)),
        ],
        out_specs=pl.BlockSpec((NC, _B_BLK, DH), lambda i: (0, i, 0)),
        out_shape=jax.ShapeDtypeStruct((NC, N, DH), jnp.float32),
    )(x, W, hist_t)


_C_CHUNK = 64
_C_PER_TILE = EPAD // NS
_C_CHUNKS = _C_PER_TILE // _C_CHUNK


_C_NBUF = 4
_C_PHALF = _C_PER_TILE // 2
_C_HCHUNKS = _C_PHALF // _C_CHUNK
_PACK_SHIFT = 14
_PACK_MASK = (1 << _PACK_SHIFT) - 1


@functools.partial(
    pl.kernel,
    out_type=jax.ShapeDtypeStruct((NC, NPAD, DH), jnp.float32),
    mesh=_sc_mesh,
    scratch_types=[
        pltpu.MemorySpace.VMEM((_C_PHALF,), jnp.int32),
    ] + [
        pltpu.MemorySpace.VMEM((_C_CHUNK,), jnp.int32)
        for _ in range(2 * _C_NBUF)
    ] + [
        pltpu.MemorySpace.VMEM((_C_CHUNK, DH), jnp.float32)
        for _ in range(_C_NBUF)
    ] + [
        pltpu.MemorySpace.VMEM_SHARED((NPAD, DH), jnp.float32),
    ] + [pltpu.SemaphoreType.DMA for _ in range(2 * _C_NBUF + 1)],
)
def _scatter_kernel(g2, packed1d, out, packedbuf, *rest):
    srcc = rest[:_C_NBUF]
    dstc = rest[_C_NBUF:2 * _C_NBUF]
    rows = rest[2 * _C_NBUF:3 * _C_NBUF]
    s_sh = rest[3 * _C_NBUF]
    semg = rest[3 * _C_NBUF + 1:4 * _C_NBUF + 1]
    sems = rest[4 * _C_NBUF + 1:5 * _C_NBUF + 1]
    semp = rest[5 * _C_NBUF + 1]
    rows0 = rows[0]
    c = lax.axis_index("c")
    s = lax.axis_index("s")
    nrows_t = NPAD // NS
    coff = c * N

    row_lo = s * nrows_t
    zvec = jnp.zeros((16,), jnp.float32)

    def _zbody(r, _):
        for q in range(DH // 16):
            rows0[r, pl.ds(q * 16, 16)] = zvec
        return 0

    lax.fori_loop(0, _C_CHUNK, _zbody, 0)
    gload = pltpu.async_copy(
        g2.at[pl.ds(coff + row_lo, 400)], s_sh.at[pl.ds(row_lo, 400)],
        semg[1])
    iload = pltpu.async_copy(
        packed1d.at[pl.ds(s * _C_PER_TILE, _C_PHALF)], packedbuf, semp)

    @pl.when(s < NS - 1)
    def _():
        pltpu.async_copy(
            g2.at[pl.ds(coff + row_lo + 400, nrows_t - 400)],
            s_sh.at[pl.ds(row_lo + 400, nrows_t - 400)], semg[2]).wait()

    @pl.when(s == NS - 1)
    def _():
        pltpu.sync_copy(rows0, s_sh.at[pl.ds(N, _C_CHUNK)])
        pltpu.sync_copy(rows0, s_sh.at[pl.ds(N + _C_CHUNK, _C_CHUNK)])
        pltpu.sync_copy(rows0, s_sh.at[pl.ds(N + 2 * _C_CHUNK, _C_CHUNK)])
        pltpu.sync_copy(
            rows0.at[pl.ds(0, 48)],
            s_sh.at[pl.ds(N + 3 * _C_CHUNK, 48)])

    gload.wait()
    iload.wait()
    plsc.subcore_barrier()

    def _unpack(k, buf):
        for i in range(_C_CHUNK // 16):
            v = packedbuf[pl.ds((k % _C_HCHUNKS) * _C_CHUNK + i * 16, 16)]
            srcc[buf][pl.ds(i * 16, 16)] = (v & _PACK_MASK) + coff
            dstc[buf][pl.ds(i * 16, 16)] = v >> _PACK_SHIFT

    g_desc = [None] * _C_NBUF
    s_desc = [None] * _C_NBUF
    reload = None
    _unpack(0, 0)
    g_desc[0] = pltpu.async_copy(g2.at[srcc[0]], rows[0], semg[0])
    for k in range(_C_CHUNKS):
        p = k % _C_NBUF
        q = (k + 1) % _C_NBUF
        if k + 1 < _C_CHUNKS:
            if s_desc[q] is not None:
                s_desc[q].wait()
            if k + 1 == _C_HCHUNKS:
                reload.wait()
            _unpack(k + 1, q)
            if k + 2 == _C_HCHUNKS:
                reload = pltpu.async_copy(
                    packed1d.at[pl.ds(s * _C_PER_TILE + _C_PHALF, _C_PHALF)],
                    packedbuf, semp)
            g_desc[q] = pltpu.async_copy(g2.at[srcc[q]], rows[q], semg[q])
        g_desc[p].wait()
        s_desc[p] = pltpu.async_copy(
            rows[p], s_sh.at[dstc[p]], sems[p], add=True)
    for d in s_desc:
        if d is not None:
            d.wait()
    plsc.subcore_barrier()

    pltpu.sync_copy(
        s_sh.at[pl.ds(s * nrows_t, nrows_t)],
        out.at[c, pl.ds(s * nrows_t, nrows_t)],
    )


def _final_body(s0_ref, s1_ref, hist_ref, b_ref, o_ref):
    deg = 1.0 + hist_ref[:, 0] + hist_ref[:, 1]
    dinv = lax.rsqrt(deg)[:, None]
    b = b_ref[...]
    lo = dinv * s0_ref[0] + b[:, :DH]
    hi = dinv * s1_ref[0] + b[:, DH:]
    o_ref[:, :DH] = jnp.maximum(lo, 0.0)
    o_ref[:, DH:] = jnp.maximum(hi, 0.0)


def _finalize(S, hist_t, b):
    return pl.pallas_call(
        _final_body,
        grid=(N // _B_BLK,),
        in_specs=[
            pl.BlockSpec((1, _B_BLK, DH), lambda i: (0, i, 0)),
            pl.BlockSpec((1, _B_BLK, DH), lambda i: (1, i, 0)),
            pl.BlockSpec((_B_BLK, NC), lambda i: (i, 0)),
            pl.BlockSpec((1, D), lambda i: (0, 0)),
        ],
        out_specs=pl.BlockSpec((_B_BLK, D), lambda i: (i, 0)),
        out_shape=jax.ShapeDtypeStruct((N, D), jnp.float32),
    )(S, S, hist_t, b)


@jax.jit
def kernel(x, edge_index, W, b):
    src = edge_index[0].astype(jnp.int32)
    dst = edge_index[1].astype(jnp.int32)
    npad = EPAD - E
    pad_src = (jnp.arange(npad, dtype=jnp.int32) * 13) % N
    pad_dst = N + (jnp.arange(npad, dtype=jnp.int32) % (NPAD - N))
    src_p = jnp.concatenate([src, pad_src])
    dst_p = jnp.concatenate([dst, pad_dst])
    dst2d = dst_p.reshape(EPAD // 128, 128)

    hist = _deg_kernel(dst2d)
    hist_t = hist.T[:N]
    g = _matmul_scale(x, W, hist_t)
    g2 = g.reshape(NC * N, DH)
    packed = src_p | (dst_p << _PACK_SHIFT)
    S = _scatter_kernel(g2, packed)
    out = _finalize(S, hist_t, b.reshape(1, D))
    return out


# speedup vs baseline: 1.0903x; 1.0903x over previous
import functools

import jax
import jax.numpy as jnp
from jax import lax
from jax.experimental import pallas as pl
from jax.experimental.pallas import tpu as pltpu
from jax.experimental.pallas import tpu_sc as plsc

N = 10000
NPAD = 10240
D = 256
DH = 128
E = 160000
EPAD = 163840
NC = 2
NS = 16

_A_ROWS = EPAD // (NC * NS) // 128

_sc_mesh = plsc.VectorSubcoreMesh(core_axis_name="c", subcore_axis_name="s")


@functools.partial(
    pl.kernel,
    out_type=jax.ShapeDtypeStruct((NC, NPAD), jnp.float32),
    mesh=_sc_mesh,
    scratch_types=[
        pltpu.MemorySpace.VMEM((_A_ROWS, 128), jnp.int32),
        pltpu.MemorySpace.VMEM((128,), jnp.float32),
        pltpu.MemorySpace.VMEM((NPAD // NS,), jnp.float32),
        pltpu.MemorySpace.VMEM_SHARED((NPAD,), jnp.float32),
        pltpu.SemaphoreType.DMA,
    ],
)
def _deg_kernel(dst2d, out, idx_v, ones_v, zero_v, hist_sh, sem):
    c = lax.axis_index("c")
    s = lax.axis_index("s")
    zvec = jnp.zeros((16,), jnp.float32)
    for i in range(NPAD // NS // 16):
        zero_v[pl.ds(i * 16, 16)] = zvec
    ovec = jnp.ones((16,), jnp.float32)
    for i in range(8):
        ones_v[pl.ds(i * 16, 16)] = ovec
    pltpu.sync_copy(zero_v, hist_sh.at[pl.ds(s * (NPAD // NS), NPAD // NS)])
    plsc.subcore_barrier()
    row0 = (c * NS + s) * _A_ROWS
    pltpu.sync_copy(dst2d.at[pl.ds(row0, _A_ROWS)], idx_v)
    descs = []
    for j in range(_A_ROWS):
        descs.append(
            pltpu.async_copy(ones_v, hist_sh.at[idx_v.at[j]], sem, add=True)
        )
    for d in descs:
        d.wait()
    plsc.subcore_barrier()
    pltpu.sync_copy(
        hist_sh.at[pl.ds(s * (NPAD // NS), NPAD // NS)],
        out.at[c, pl.ds(s * (NPAD // NS), NPAD // NS)],
    )


_B_BLK = 1000


def _matmul_body(x_ref, w_ref, hist_ref, g_ref):
    h = jnp.dot(x_ref[...], w_ref[...], preferred_element_type=jnp.float32)
    deg = 1.0 + hist_ref[:, 0] + hist_ref[:, 1]
    dinv = lax.rsqrt(deg)[:, None]
    g_ref[0] = h[:, :DH] * dinv
    g_ref[1] = h[:, DH:] * dinv


def _matmul_scale(x, W, hist_t):
    return pl.pallas_call(
        _matmul_body,
        grid=(N // _B_BLK,),
        in_specs=[
            pl.BlockSpec((_B_BLK, D), lambda i: (i, 0)),
            pl.BlockSpec((D, D), lambda i: (0, 0)),
            pl.BlockSpec((_B_BLK, NC), lambda i: (i, 0)),
        ],
        out_specs=pl.BlockSpec((NC, _B_BLK, DH), lambda i: (0, i, 0)),
        out_shape=jax.ShapeDtypeStruct((NC, N, DH), jnp.float32),
    )(x, W, hist_t)


_C_CHUNK = 128
_C_PER_TILE = EPAD // NS
_C_CHUNKS = _C_PER_TILE // _C_CHUNK


_C_NBUF = 2
_C_PHALF = _C_PER_TILE // 2
_C_HCHUNKS = _C_PHALF // _C_CHUNK
_PACK_SHIFT = 14
_PACK_MASK = (1 << _PACK_SHIFT) - 1


@functools.partial(
    pl.kernel,
    out_type=jax.ShapeDtypeStruct((NC, NPAD, DH), jnp.float32),
    mesh=_sc_mesh,
    scratch_types=[
        pltpu.MemorySpace.VMEM((_C_PHALF,), jnp.int32),
    ] + [
        pltpu.MemorySpace.VMEM((_C_CHUNK,), jnp.int32)
        for _ in range(2 * _C_NBUF)
    ] + [
        pltpu.MemorySpace.VMEM((_C_CHUNK, DH), jnp.float32)
        for _ in range(_C_NBUF)
    ] + [
        pltpu.MemorySpace.VMEM_SHARED((NPAD, DH), jnp.float32),
    ] + [pltpu.SemaphoreType.DMA for _ in range(2 * _C_NBUF + 1)],
)
def _scatter_kernel(g2, packed1d, out, packedbuf, *rest):
    srcc = rest[:_C_NBUF]
    dstc = rest[_C_NBUF:2 * _C_NBUF]
    rows = rest[2 * _C_NBUF:3 * _C_NBUF]
    s_sh = rest[3 * _C_NBUF]
    semg = rest[3 * _C_NBUF + 1:4 * _C_NBUF + 1]
    sems = rest[4 * _C_NBUF + 1:5 * _C_NBUF + 1]
    semp = rest[5 * _C_NBUF + 1]
    rows0 = rows[0]
    c = lax.axis_index("c")
    s = lax.axis_index("s")
    nrows_t = NPAD // NS
    coff = c * N

    row_lo = s * nrows_t
    zvec = jnp.zeros((16,), jnp.float32)

    def _zbody(r, _):
        for q in range(DH // 16):
            rows0[r, pl.ds(q * 16, 16)] = zvec
        return 0

    lax.fori_loop(0, _C_CHUNK, _zbody, 0)
    gload = pltpu.async_copy(
        g2.at[pl.ds(coff + row_lo, 400)], s_sh.at[pl.ds(row_lo, 400)],
        semg[1])
    iload = pltpu.async_copy(
        packed1d.at[pl.ds(s * _C_PER_TILE, _C_PHALF)], packedbuf, semp)

    @pl.when(s < NS - 1)
    def _():
        pltpu.async_copy(
            g2.at[pl.ds(coff + row_lo + 400, nrows_t - 400)],
            s_sh.at[pl.ds(row_lo + 400, nrows_t - 400)], sems[0]).wait()

    @pl.when(s == NS - 1)
    def _():
        pltpu.sync_copy(rows0, s_sh.at[pl.ds(N, _C_CHUNK)])
        pltpu.sync_copy(
            rows0.at[pl.ds(0, NPAD - N - _C_CHUNK)],
            s_sh.at[pl.ds(N + _C_CHUNK, NPAD - N - _C_CHUNK)])

    gload.wait()
    iload.wait()
    plsc.subcore_barrier()

    def _unpack(k, buf):
        for i in range(_C_CHUNK // 16):
            v = packedbuf[pl.ds((k % _C_HCHUNKS) * _C_CHUNK + i * 16, 16)]
            srcc[buf][pl.ds(i * 16, 16)] = (v & _PACK_MASK) + coff
            dstc[buf][pl.ds(i * 16, 16)] = v >> _PACK_SHIFT

    g_desc = [None] * _C_NBUF
    s_desc = [None] * _C_NBUF
    reload = None
    _unpack(0, 0)
    g_desc[0] = pltpu.async_copy(g2.at[srcc[0]], rows[0], semg[0])
    for k in range(_C_CHUNKS):
        p = k % _C_NBUF
        q = (k + 1) % _C_NBUF
        if k + 1 < _C_CHUNKS:
            if s_desc[q] is not None:
                s_desc[q].wait()
            if k + 1 == _C_HCHUNKS:
                reload.wait()
            _unpack(k + 1, q)
            if k + 2 == _C_HCHUNKS:
                reload = pltpu.async_copy(
                    packed1d.at[pl.ds(s * _C_PER_TILE + _C_PHALF, _C_PHALF)],
                    packedbuf, semp)
            g_desc[q] = pltpu.async_copy(g2.at[srcc[q]], rows[q], semg[q])
        g_desc[p].wait()
        s_desc[p] = pltpu.async_copy(
            rows[p], s_sh.at[dstc[p]], sems[p], add=True)
    for d in s_desc:
        if d is not None:
            d.wait()
    plsc.subcore_barrier()

    pltpu.sync_copy(
        s_sh.at[pl.ds(s * nrows_t, nrows_t)],
        out.at[c, pl.ds(s * nrows_t, nrows_t)],
    )


def _final_body(s0_ref, s1_ref, hist_ref, b_ref, o_ref):
    deg = 1.0 + hist_ref[:, 0] + hist_ref[:, 1]
    dinv = lax.rsqrt(deg)[:, None]
    b = b_ref[...]
    lo = dinv * s0_ref[0] + b[:, :DH]
    hi = dinv * s1_ref[0] + b[:, DH:]
    o_ref[:, :DH] = jnp.maximum(lo, 0.0)
    o_ref[:, DH:] = jnp.maximum(hi, 0.0)


def _finalize(S, hist_t, b):
    return pl.pallas_call(
        _final_body,
        grid=(N // _B_BLK,),
        in_specs=[
            pl.BlockSpec((1, _B_BLK, DH), lambda i: (0, i, 0)),
            pl.BlockSpec((1, _B_BLK, DH), lambda i: (1, i, 0)),
            pl.BlockSpec((_B_BLK, NC), lambda i: (i, 0)),
            pl.BlockSpec((1, D), lambda i: (0, 0)),
        ],
        out_specs=pl.BlockSpec((_B_BLK, D), lambda i: (i, 0)),
        out_shape=jax.ShapeDtypeStruct((N, D), jnp.float32),
    )(S, S, hist_t, b)


@jax.jit
def kernel(x, edge_index, W, b):
    src = edge_index[0].astype(jnp.int32)
    dst = edge_index[1].astype(jnp.int32)
    npad = EPAD - E
    pad_src = (jnp.arange(npad, dtype=jnp.int32) * 13) % N
    pad_dst = N + (jnp.arange(npad, dtype=jnp.int32) % (NPAD - N))
    src_p = jnp.concatenate([src, pad_src])
    dst_p = jnp.concatenate([dst, pad_dst])
    dst2d = dst_p.reshape(EPAD // 128, 128)

    hist = _deg_kernel(dst2d)
    hist_t = hist.T[:N]
    g = _matmul_scale(x, W, hist_t)
    g2 = g.reshape(NC * N, DH)
    packed = src_p | (dst_p << _PACK_SHIFT)
    S = _scatter_kernel(g2, packed)
    out = _finalize(S, hist_t, b.reshape(1, D))
    return out


# speedup vs baseline: 1.1085x; 1.0167x over previous
import functools

import jax
import jax.numpy as jnp
from jax import lax
from jax.experimental import pallas as pl
from jax.experimental.pallas import tpu as pltpu
from jax.experimental.pallas import tpu_sc as plsc

N = 10000
NPAD = 10240
D = 256
DH = 128
E = 160000
EPAD = 163840
NC = 2
NS = 16

_A_ROWS = EPAD // (NC * NS) // 128

_sc_mesh = plsc.VectorSubcoreMesh(core_axis_name="c", subcore_axis_name="s")


@functools.partial(
    pl.kernel,
    out_type=jax.ShapeDtypeStruct((NC, NPAD), jnp.float32),
    mesh=_sc_mesh,
    scratch_types=[
        pltpu.MemorySpace.VMEM((_A_ROWS, 128), jnp.int32),
        pltpu.MemorySpace.VMEM((128,), jnp.float32),
        pltpu.MemorySpace.VMEM((NPAD // NS,), jnp.float32),
        pltpu.MemorySpace.VMEM_SHARED((NPAD,), jnp.float32),
        pltpu.SemaphoreType.DMA,
    ],
)
def _deg_kernel(dst2d, out, idx_v, ones_v, zero_v, hist_sh, sem):
    c = lax.axis_index("c")
    s = lax.axis_index("s")
    zvec = jnp.zeros((16,), jnp.float32)
    for i in range(NPAD // NS // 16):
        zero_v[pl.ds(i * 16, 16)] = zvec
    ovec = jnp.ones((16,), jnp.float32)
    for i in range(8):
        ones_v[pl.ds(i * 16, 16)] = ovec
    pltpu.sync_copy(zero_v, hist_sh.at[pl.ds(s * (NPAD // NS), NPAD // NS)])
    plsc.subcore_barrier()
    row0 = (c * NS + s) * _A_ROWS
    pltpu.sync_copy(dst2d.at[pl.ds(row0, _A_ROWS)], idx_v)
    descs = []
    for j in range(_A_ROWS):
        descs.append(
            pltpu.async_copy(ones_v, hist_sh.at[idx_v.at[j]], sem, add=True)
        )
    for d in descs:
        d.wait()
    plsc.subcore_barrier()
    pltpu.sync_copy(
        hist_sh.at[pl.ds(s * (NPAD // NS), NPAD // NS)],
        out.at[c, pl.ds(s * (NPAD // NS), NPAD // NS)],
    )


_B_BLK = 2000


def _matmul_body(x_ref, w_ref, hist_ref, g_ref):
    h = jnp.dot(x_ref[...], w_ref[...], preferred_element_type=jnp.float32)
    deg = 1.0 + hist_ref[:, 0] + hist_ref[:, 1]
    dinv = lax.rsqrt(deg)[:, None]
    g_ref[0] = h[:, :DH] * dinv
    g_ref[1] = h[:, DH:] * dinv


def _matmul_scale(x, W, hist_t):
    return pl.pallas_call(
        _matmul_body,
        grid=(N // _B_BLK,),
        in_specs=[
            pl.BlockSpec((_B_BLK, D), lambda i: (i, 0)),
            pl.BlockSpec((D, D), lambda i: (0, 0)),
            pl.BlockSpec((_B_BLK, NC), lambda i: (i, 0)),
        ],
        out_specs=pl.BlockSpec((NC, _B_BLK, DH), lambda i: (0, i, 0)),
        out_shape=jax.ShapeDtypeStruct((NC, N, DH), jnp.float32),
    )(x, W, hist_t)


_C_CHUNK = 128
_C_PER_TILE = EPAD // NS
_C_CHUNKS = _C_PER_TILE // _C_CHUNK


_C_NBUF = 2
_C_PHALF = _C_PER_TILE // 2
_C_HCHUNKS = _C_PHALF // _C_CHUNK
_PACK_SHIFT = 14
_PACK_MASK = (1 << _PACK_SHIFT) - 1


@functools.partial(
    pl.kernel,
    out_type=jax.ShapeDtypeStruct((NC, NPAD, DH), jnp.float32),
    mesh=_sc_mesh,
    scratch_types=[
        pltpu.MemorySpace.VMEM((_C_PHALF,), jnp.int32),
    ] + [
        pltpu.MemorySpace.VMEM((_C_CHUNK,), jnp.int32)
        for _ in range(2 * _C_NBUF)
    ] + [
        pltpu.MemorySpace.VMEM((_C_CHUNK, DH), jnp.float32)
        for _ in range(_C_NBUF)
    ] + [
        pltpu.MemorySpace.VMEM_SHARED((NPAD, DH), jnp.float32),
    ] + [pltpu.SemaphoreType.DMA for _ in range(2 * _C_NBUF + 1)],
)
def _scatter_kernel(g2, packed1d, out, packedbuf, *rest):
    srcc = rest[:_C_NBUF]
    dstc = rest[_C_NBUF:2 * _C_NBUF]
    rows = rest[2 * _C_NBUF:3 * _C_NBUF]
    s_sh = rest[3 * _C_NBUF]
    semg = rest[3 * _C_NBUF + 1:4 * _C_NBUF + 1]
    sems = rest[4 * _C_NBUF + 1:5 * _C_NBUF + 1]
    semp = rest[5 * _C_NBUF + 1]
    rows0 = rows[0]
    c = lax.axis_index("c")
    s = lax.axis_index("s")
    nrows_t = NPAD // NS
    coff = c * N

    row_lo = s * nrows_t
    zvec = jnp.zeros((16,), jnp.float32)

    def _zbody(r, _):
        for q in range(DH // 16):
            rows0[r, pl.ds(q * 16, 16)] = zvec
        return 0

    lax.fori_loop(0, _C_CHUNK, _zbody, 0)
    gload = pltpu.async_copy(
        g2.at[pl.ds(coff + row_lo, 400)], s_sh.at[pl.ds(row_lo, 400)],
        semg[1])
    iload = pltpu.async_copy(
        packed1d.at[pl.ds(s * _C_PER_TILE, _C_PHALF)], packedbuf, semp)

    @pl.when(s < NS - 1)
    def _():
        pltpu.async_copy(
            g2.at[pl.ds(coff + row_lo + 400, nrows_t - 400)],
            s_sh.at[pl.ds(row_lo + 400, nrows_t - 400)], sems[0]).wait()

    @pl.when(s == NS - 1)
    def _():
        pltpu.sync_copy(rows0, s_sh.at[pl.ds(N, _C_CHUNK)])
        pltpu.sync_copy(
            rows0.at[pl.ds(0, NPAD - N - _C_CHUNK)],
            s_sh.at[pl.ds(N + _C_CHUNK, NPAD - N - _C_CHUNK)])

    gload.wait()
    iload.wait()
    plsc.subcore_barrier()

    def _unpack(k, buf):
        for i in range(_C_CHUNK // 16):
            v = packedbuf[pl.ds((k % _C_HCHUNKS) * _C_CHUNK + i * 16, 16)]
            srcc[buf][pl.ds(i * 16, 16)] = (v & _PACK_MASK) + coff
            dstc[buf][pl.ds(i * 16, 16)] = v >> _PACK_SHIFT

    g_desc = [None] * _C_NBUF
    s_desc = [None] * _C_NBUF
    reload = None
    _unpack(0, 0)
    g_desc[0] = pltpu.async_copy(g2.at[srcc[0]], rows[0], semg[0])
    for k in range(_C_CHUNKS):
        p = k % _C_NBUF
        q = (k + 1) % _C_NBUF
        if k + 1 < _C_CHUNKS:
            if s_desc[q] is not None:
                s_desc[q].wait()
            if k + 1 == _C_HCHUNKS:
                reload.wait()
            _unpack(k + 1, q)
            if k + 2 == _C_HCHUNKS:
                reload = pltpu.async_copy(
                    packed1d.at[pl.ds(s * _C_PER_TILE + _C_PHALF, _C_PHALF)],
                    packedbuf, semp)
            g_desc[q] = pltpu.async_copy(g2.at[srcc[q]], rows[q], semg[q])
        g_desc[p].wait()
        s_desc[p] = pltpu.async_copy(
            rows[p], s_sh.at[dstc[p]], sems[p], add=True)
    for d in s_desc:
        if d is not None:
            d.wait()
    plsc.subcore_barrier()

    pltpu.sync_copy(
        s_sh.at[pl.ds(s * nrows_t, nrows_t)],
        out.at[c, pl.ds(s * nrows_t, nrows_t)],
    )


def _final_body(s0_ref, s1_ref, hist_ref, b_ref, o_ref):
    deg = 1.0 + hist_ref[:, 0] + hist_ref[:, 1]
    dinv = lax.rsqrt(deg)[:, None]
    b = b_ref[...]
    lo = dinv * s0_ref[0] + b[:, :DH]
    hi = dinv * s1_ref[0] + b[:, DH:]
    o_ref[:, :DH] = jnp.maximum(lo, 0.0)
    o_ref[:, DH:] = jnp.maximum(hi, 0.0)


def _finalize(S, hist_t, b):
    return pl.pallas_call(
        _final_body,
        grid=(N // _B_BLK,),
        in_specs=[
            pl.BlockSpec((1, _B_BLK, DH), lambda i: (0, i, 0)),
            pl.BlockSpec((1, _B_BLK, DH), lambda i: (1, i, 0)),
            pl.BlockSpec((_B_BLK, NC), lambda i: (i, 0)),
            pl.BlockSpec((1, D), lambda i: (0, 0)),
        ],
        out_specs=pl.BlockSpec((_B_BLK, D), lambda i: (i, 0)),
        out_shape=jax.ShapeDtypeStruct((N, D), jnp.float32),
    )(S, S, hist_t, b)


@jax.jit
def kernel(x, edge_index, W, b):
    src = edge_index[0].astype(jnp.int32)
    dst = edge_index[1].astype(jnp.int32)
    npad = EPAD - E
    pad_src = (jnp.arange(npad, dtype=jnp.int32) * 13) % N
    pad_dst = N + (jnp.arange(npad, dtype=jnp.int32) % (NPAD - N))
    src_p = jnp.concatenate([src, pad_src])
    dst_p = jnp.concatenate([dst, pad_dst])
    dst2d = dst_p.reshape(EPAD // 128, 128)

    hist = _deg_kernel(dst2d)
    hist_t = hist.T[:N]
    g = _matmul_scale(x, W, hist_t)
    g2 = g.reshape(NC * N, DH)
    packed = src_p | (dst_p << _PACK_SHIFT)
    S = _scatter_kernel(g2, packed)
    out = _finalize(S, hist_t, b.reshape(1, D))
    return out


# speedup vs baseline: 1.1374x; 1.0261x over previous
import functools

import jax
import jax.numpy as jnp
from jax import lax
from jax.experimental import pallas as pl
from jax.experimental.pallas import tpu as pltpu
from jax.experimental.pallas import tpu_sc as plsc

N = 10000
NPAD = 10240
D = 256
DH = 128
E = 160000
EPAD = 163840
NC = 2
NS = 16

_A_ROWS = EPAD // (NC * NS) // 128

_sc_mesh = plsc.VectorSubcoreMesh(core_axis_name="c", subcore_axis_name="s")


@functools.partial(
    pl.kernel,
    out_type=jax.ShapeDtypeStruct((NC, NPAD), jnp.float32),
    mesh=_sc_mesh,
    scratch_types=[
        pltpu.MemorySpace.VMEM((_A_ROWS, 128), jnp.int32),
        pltpu.MemorySpace.VMEM((128,), jnp.float32),
        pltpu.MemorySpace.VMEM((NPAD // NS,), jnp.float32),
        pltpu.MemorySpace.VMEM_SHARED((NPAD,), jnp.float32),
        pltpu.SemaphoreType.DMA,
    ],
)
def _deg_kernel(dst2d, out, idx_v, ones_v, zero_v, hist_sh, sem):
    c = lax.axis_index("c")
    s = lax.axis_index("s")
    zvec = jnp.zeros((16,), jnp.float32)
    for i in range(NPAD // NS // 16):
        zero_v[pl.ds(i * 16, 16)] = zvec
    ovec = jnp.ones((16,), jnp.float32)
    for i in range(8):
        ones_v[pl.ds(i * 16, 16)] = ovec
    pltpu.sync_copy(zero_v, hist_sh.at[pl.ds(s * (NPAD // NS), NPAD // NS)])
    plsc.subcore_barrier()
    row0 = (c * NS + s) * _A_ROWS
    pltpu.sync_copy(dst2d.at[pl.ds(row0, _A_ROWS)], idx_v)
    descs = []
    for j in range(_A_ROWS):
        descs.append(
            pltpu.async_copy(ones_v, hist_sh.at[idx_v.at[j]], sem, add=True)
        )
    for d in descs:
        d.wait()
    plsc.subcore_barrier()
    pltpu.sync_copy(
        hist_sh.at[pl.ds(s * (NPAD // NS), NPAD // NS)],
        out.at[c, pl.ds(s * (NPAD // NS), NPAD // NS)],
    )


_B_BLK = 5000


def _matmul_body(x_ref, w_ref, hist_ref, g_ref):
    h = jnp.dot(x_ref[...], w_ref[...], preferred_element_type=jnp.float32)
    deg = 1.0 + hist_ref[:, 0] + hist_ref[:, 1]
    dinv = lax.rsqrt(deg)[:, None]
    g_ref[0] = h[:, :DH] * dinv
    g_ref[1] = h[:, DH:] * dinv


def _matmul_scale(x, W, hist_t):
    return pl.pallas_call(
        _matmul_body,
        grid=(N // _B_BLK,),
        in_specs=[
            pl.BlockSpec((_B_BLK, D), lambda i: (i, 0)),
            pl.BlockSpec((D, D), lambda i: (0, 0)),
            pl.BlockSpec((_B_BLK, NC), lambda i: (i, 0)),
        ],
        out_specs=pl.BlockSpec((NC, _B_BLK, DH), lambda i: (0, i, 0)),
        out_shape=jax.ShapeDtypeStruct((NC, N, DH), jnp.float32),
    )(x, W, hist_t)


_C_CHUNK = 128
_C_PER_TILE = EPAD // NS
_C_CHUNKS = _C_PER_TILE // _C_CHUNK


_C_NBUF = 2
_C_PHALF = _C_PER_TILE // 2
_C_HCHUNKS = _C_PHALF // _C_CHUNK
_PACK_SHIFT = 14
_PACK_MASK = (1 << _PACK_SHIFT) - 1


@functools.partial(
    pl.kernel,
    out_type=jax.ShapeDtypeStruct((NC, NPAD, DH), jnp.float32),
    mesh=_sc_mesh,
    scratch_types=[
        pltpu.MemorySpace.VMEM((_C_PHALF,), jnp.int32),
    ] + [
        pltpu.MemorySpace.VMEM((_C_CHUNK,), jnp.int32)
        for _ in range(2 * _C_NBUF)
    ] + [
        pltpu.MemorySpace.VMEM((_C_CHUNK, DH), jnp.float32)
        for _ in range(_C_NBUF)
    ] + [
        pltpu.MemorySpace.VMEM_SHARED((NPAD, DH), jnp.float32),
    ] + [pltpu.SemaphoreType.DMA for _ in range(2 * _C_NBUF + 1)],
)
def _scatter_kernel(g2, packed1d, out, packedbuf, *rest):
    srcc = rest[:_C_NBUF]
    dstc = rest[_C_NBUF:2 * _C_NBUF]
    rows = rest[2 * _C_NBUF:3 * _C_NBUF]
    s_sh = rest[3 * _C_NBUF]
    semg = rest[3 * _C_NBUF + 1:4 * _C_NBUF + 1]
    sems = rest[4 * _C_NBUF + 1:5 * _C_NBUF + 1]
    semp = rest[5 * _C_NBUF + 1]
    rows0 = rows[0]
    c = lax.axis_index("c")
    s = lax.axis_index("s")
    nrows_t = NPAD // NS
    coff = c * N

    row_lo = s * nrows_t
    zvec = jnp.zeros((16,), jnp.float32)

    def _zbody(r, _):
        for q in range(DH // 16):
            rows0[r, pl.ds(q * 16, 16)] = zvec
        return 0

    lax.fori_loop(0, _C_CHUNK, _zbody, 0)
    gload = pltpu.async_copy(
        g2.at[pl.ds(coff + row_lo, 400)], s_sh.at[pl.ds(row_lo, 400)],
        semg[1])
    iload = pltpu.async_copy(
        packed1d.at[pl.ds(s * _C_PER_TILE, _C_PHALF)], packedbuf, semp)

    @pl.when(s < NS - 1)
    def _():
        pltpu.async_copy(
            g2.at[pl.ds(coff + row_lo + 400, nrows_t - 400)],
            s_sh.at[pl.ds(row_lo + 400, nrows_t - 400)], sems[0]).wait()

    @pl.when(s == NS - 1)
    def _():
        pltpu.sync_copy(rows0, s_sh.at[pl.ds(N, _C_CHUNK)])
        pltpu.sync_copy(
            rows0.at[pl.ds(0, NPAD - N - _C_CHUNK)],
            s_sh.at[pl.ds(N + _C_CHUNK, NPAD - N - _C_CHUNK)])

    gload.wait()
    iload.wait()
    plsc.subcore_barrier()

    def _unpack(k, buf):
        for i in range(_C_CHUNK // 16):
            v = packedbuf[pl.ds((k % _C_HCHUNKS) * _C_CHUNK + i * 16, 16)]
            srcc[buf][pl.ds(i * 16, 16)] = (v & _PACK_MASK) + coff
            dstc[buf][pl.ds(i * 16, 16)] = v >> _PACK_SHIFT

    g_desc = [None] * _C_NBUF
    s_desc = [None] * _C_NBUF
    reload = None
    _unpack(0, 0)
    g_desc[0] = pltpu.async_copy(g2.at[srcc[0]], rows[0], semg[0])
    for k in range(_C_CHUNKS):
        p = k % _C_NBUF
        q = (k + 1) % _C_NBUF
        if k + 1 < _C_CHUNKS:
            if s_desc[q] is not None:
                s_desc[q].wait()
            if k + 1 == _C_HCHUNKS:
                reload.wait()
            _unpack(k + 1, q)
            if k + 2 == _C_HCHUNKS:
                reload = pltpu.async_copy(
                    packed1d.at[pl.ds(s * _C_PER_TILE + _C_PHALF, _C_PHALF)],
                    packedbuf, semp)
            g_desc[q] = pltpu.async_copy(g2.at[srcc[q]], rows[q], semg[q])
        g_desc[p].wait()
        s_desc[p] = pltpu.async_copy(
            rows[p], s_sh.at[dstc[p]], sems[p], add=True)
    for d in s_desc:
        if d is not None:
            d.wait()
    plsc.subcore_barrier()

    pltpu.sync_copy(
        s_sh.at[pl.ds(s * nrows_t, nrows_t)],
        out.at[c, pl.ds(s * nrows_t, nrows_t)],
    )


def _final_body(s0_ref, s1_ref, hist_ref, b_ref, o_ref):
    deg = 1.0 + hist_ref[:, 0] + hist_ref[:, 1]
    dinv = lax.rsqrt(deg)[:, None]
    b = b_ref[...]
    lo = dinv * s0_ref[0] + b[:, :DH]
    hi = dinv * s1_ref[0] + b[:, DH:]
    o_ref[:, :DH] = jnp.maximum(lo, 0.0)
    o_ref[:, DH:] = jnp.maximum(hi, 0.0)


def _finalize(S, hist_t, b):
    return pl.pallas_call(
        _final_body,
        grid=(N // _B_BLK,),
        in_specs=[
            pl.BlockSpec((1, _B_BLK, DH), lambda i: (0, i, 0)),
            pl.BlockSpec((1, _B_BLK, DH), lambda i: (1, i, 0)),
            pl.BlockSpec((_B_BLK, NC), lambda i: (i, 0)),
            pl.BlockSpec((1, D), lambda i: (0, 0)),
        ],
        out_specs=pl.BlockSpec((_B_BLK, D), lambda i: (i, 0)),
        out_shape=jax.ShapeDtypeStruct((N, D), jnp.float32),
    )(S, S, hist_t, b)


@jax.jit
def kernel(x, edge_index, W, b):
    src = edge_index[0].astype(jnp.int32)
    dst = edge_index[1].astype(jnp.int32)
    npad = EPAD - E
    pad_src = (jnp.arange(npad, dtype=jnp.int32) * 13) % N
    pad_dst = N + (jnp.arange(npad, dtype=jnp.int32) % (NPAD - N))
    src_p = jnp.concatenate([src, pad_src])
    dst_p = jnp.concatenate([dst, pad_dst])
    dst2d = dst_p.reshape(EPAD // 128, 128)

    hist = _deg_kernel(dst2d)
    hist_t = hist.T[:N]
    g = _matmul_scale(x, W, hist_t)
    g2 = g.reshape(NC * N, DH)
    packed = src_p | (dst_p << _PACK_SHIFT)
    S = _scatter_kernel(g2, packed)
    out = _finalize(S, hist_t, b.reshape(1, D))
    return out
